# baseline (speedup 1.0000x reference)
import functools

import jax
import jax.numpy as jnp
from jax import lax
from jax.experimental import pallas as pl
from jax.experimental.pallas import tpu as pltpu
from jax.experimental.pallas import tpu_sc as plsc

_LANES = 16
_HALF = 100


def _make_pool(B, S, D, nc, ns):
  NW = nc * ns
  BPW = B // NW
  assert S == 2 * _HALF and D % _LANES == 0 and B % NW == 0
  nchunks = D // _LANES
  scale = 1.0 / S
  mesh = plsc.VectorSubcoreMesh(core_axis_name="c", subcore_axis_name="s")

  def body(x_hbm, table_hbm, out_hbm, idx_v, rows_v, out_v, sem0, sem1):
    wid = lax.axis_index("s") * nc + lax.axis_index("c")
    base = wid * BPW
    pltpu.sync_copy(x_hbm.at[pl.ds(base * 2, BPW * 2)], idx_v)
    sems = (sem0, sem1)

    def start(i, buf, sem):
      pltpu.async_copy(table_hbm.at[idx_v.at[2 * i]],
                       rows_v.at[buf, pl.ds(0, _HALF)], sem)
      pltpu.async_copy(table_hbm.at[idx_v.at[2 * i + 1]],
                       rows_v.at[buf, pl.ds(_HALF, _HALF)], sem)

    def wait(i, buf, sem):
      pltpu.make_async_copy(table_hbm.at[idx_v.at[2 * i]],
                            rows_v.at[buf, pl.ds(0, _HALF)], sem).wait()
      pltpu.make_async_copy(table_hbm.at[idx_v.at[2 * i + 1]],
                            rows_v.at[buf, pl.ds(_HALF, _HALF)], sem).wait()

    start(0, 0, sem0)

    def outer(i2, carry):
      for b in (0, 1):
        i = i2 * 2 + b
        nxt = i + 1

        @pl.when(nxt < BPW)
        def _():
          start(nxt, 1 - b, sems[1 - b])

        wait(i, b, sems[b])

        def rbody(r, acc):
          return tuple(acc[c] + rows_v[b, r, pl.ds(c * _LANES, _LANES)]
                       for c in range(nchunks))

        zero = jnp.zeros((_LANES,), jnp.float32)
        acc = lax.fori_loop(0, S, rbody, (zero,) * nchunks)
        for c in range(nchunks):
          out_v[i, pl.ds(c * _LANES, _LANES)] = acc[c] * scale
      return carry

    lax.fori_loop(0, BPW // 2, outer, 0)
    pltpu.sync_copy(out_v, out_hbm.at[pl.ds(base, BPW)])

  return pl.kernel(
      body,
      out_type=jax.ShapeDtypeStruct((B, D), jnp.float32),
      mesh=mesh,
      scratch_types=[
          pltpu.VMEM((2 * BPW, _HALF), jnp.int32),
          pltpu.VMEM((2, S, D), jnp.float32),
          pltpu.VMEM((BPW, D), jnp.float32),
          pltpu.SemaphoreType.DMA,
          pltpu.SemaphoreType.DMA,
      ],
      compiler_params=pltpu.CompilerParams(use_tc_tiling_on_sc=False),
  )


def _mlp_body(p_ref, w1_ref, b1_ref, w2_ref, b2_ref, o_ref):
  h = jnp.dot(p_ref[...], w1_ref[...], preferred_element_type=jnp.float32)
  h = jnp.maximum(h + b1_ref[...], 0.0)
  logits = jnp.dot(h, w2_ref[...], preferred_element_type=jnp.float32)
  logits = logits + b2_ref[...]
  m = jnp.max(logits, axis=1, keepdims=True)
  e = jnp.exp(logits - m)
  o_ref[...] = e / jnp.sum(e, axis=1, keepdims=True)


def kernel(x, table, W1, b1, W2, b2):
  B, S = x.shape
  V, D = table.shape
  H = W1.shape[1]
  C = W2.shape[1]

  info = plsc.get_sparse_core_info()
  pool = _make_pool(B, S, D, info.num_cores, info.num_subcores)
  pooled = pool(x.reshape(B * 2, S // 2), table)

  CP = 128
  W2p = jnp.zeros((H, CP), jnp.float32).at[:, :C].set(W2)
  b2p = jnp.full((1, CP), -1e30, jnp.float32).at[0, :C].set(b2)
  BLK = 1024
  out = pl.pallas_call(
      _mlp_body,
      out_shape=jax.ShapeDtypeStruct((B, CP), jnp.float32),
      grid=(B // BLK,),
      in_specs=[
          pl.BlockSpec((BLK, D), lambda i: (i, 0)),
          pl.BlockSpec((D, H), lambda i: (0, 0)),
          pl.BlockSpec((1, H), lambda i: (0, 0)),
          pl.BlockSpec((H, CP), lambda i: (0, 0)),
          pl.BlockSpec((1, CP), lambda i: (0, 0)),
      ],
      out_specs=pl.BlockSpec((BLK, CP), lambda i: (i, 0)),
  )(pooled, W1, b1.reshape(1, H), W2p, b2p)
  return out[:, :C]
